# baseline (speedup 1.0000x reference)
import functools

import jax
import jax.numpy as jnp
from jax import lax
from jax.experimental import pallas as pl
from jax.experimental.pallas import tpu as pltpu
from jax.experimental.pallas import tpu_sc as plsc

DIM = 32
B = 16384
NROWS = 1000000
NC = 2
NS = 16
NW = NC * NS
B_PER_W = B // NW
NBUF = 16
NQ = B_PER_W // NBUF

_mesh = plsc.VectorSubcoreMesh(core_axis_name="c", subcore_axis_name="s")


@functools.partial(
    pl.kernel,
    mesh=_mesh,
    out_type=(
        jax.ShapeDtypeStruct((DIM, B), jnp.float32),
        jax.ShapeDtypeStruct((DIM, B), jnp.float32),
        jax.ShapeDtypeStruct((DIM, B), jnp.float32),
    ),
    scratch_types=[
        pltpu.VMEM((B_PER_W,), jnp.int32),
        pltpu.VMEM((B_PER_W,), jnp.int32),
        pltpu.VMEM((B_PER_W,), jnp.int32),
        pltpu.VMEM((NBUF, DIM, 128), jnp.float32),
        pltpu.VMEM((DIM, B_PER_W), jnp.float32),
        pltpu.SemaphoreType.DMA((NBUF,)),
    ],
    compiler_params=pltpu.CompilerParams(needs_layout_passes=False),
)
def _sc_gather3(user_h, pos_h, neg_h, tabTu_h, tabTi_h,
                outTu, outTp, outTn, idx_u, idx_p, idx_n, ring_v, stage_v, sems):
    cid = lax.axis_index("c")
    sid = lax.axis_index("s")
    wid = sid * NC + cid
    base = wid * B_PER_W

    idx_srcs = (user_h, pos_h, neg_h)
    idx_bufs = (idx_u, idx_p, idx_n)
    tabs = (tabTu_h, tabTi_h, tabTi_h)
    outs = (outTu, outTp, outTn)

    for g in range(3):
        pltpu.sync_copy(idx_srcs[g].at[pl.ds(base, B_PER_W)], idx_bufs[g])

    d_lo = lax.iota(jnp.int32, 16)
    d_hi = d_lo + 16

    def _issue(tab, ix, j):
        jcol = pl.multiple_of((ix >> 7) << 7, 128)
        pltpu.async_copy(tab.at[pl.ds(0, 16), pl.ds(jcol, 128)],
                         ring_v.at[j, pl.ds(0, 16)], sems.at[j])
        pltpu.async_copy(tab.at[pl.ds(16, 16), pl.ds(jcol, 128)],
                         ring_v.at[j, pl.ds(16, 16)], sems.at[j])

    def _wait(j):
        pltpu.make_async_copy(
            tabs[0].at[pl.ds(0, 16), pl.ds(0, 128)],
            ring_v.at[j, pl.ds(0, 16)], sems.at[j],
        ).wait()
        pltpu.make_async_copy(
            tabs[0].at[pl.ds(16, 16), pl.ds(0, 128)],
            ring_v.at[j, pl.ds(16, 16)], sems.at[j],
        ).wait()

    def _extract(vec, j, k):
        ix = vec[j]
        lane = jnp.broadcast_to(ix & 127, (16,))
        pos = jnp.broadcast_to(k, (16,))
        v_lo = plsc.load_gather(ring_v.at[j], [d_lo, lane])
        v_hi = plsc.load_gather(ring_v.at[j], [d_hi, lane])
        plsc.store_scatter(stage_v, [d_lo, pos], v_lo)
        plsc.store_scatter(stage_v, [d_hi, pos], v_hi)

    for g in range(3):
        tab = tabs[g]

        idx_buf = idx_bufs[g]

        def body(q, prev_vec, tab=tab, idx_buf=idx_buf):
            vec = idx_buf[pl.ds(q * NBUF, NBUF)]

            @pl.when(q > 0)
            def _():
                for j in range(NBUF):
                    _wait(j)
                    _extract(prev_vec, j, (q - 1) * NBUF + j)

            for j in range(NBUF):
                _issue(tab, vec[j], j)
            return vec

        last_vec = lax.fori_loop(0, NQ, body, jnp.zeros((NBUF,), jnp.int32))
        for j in range(NBUF):
            _wait(j)
            _extract(last_vec, j, (NQ - 1) * NBUF + j)

        pltpu.sync_copy(stage_v, outs[g].at[:, pl.ds(base, B_PER_W)])


def kernel(user, pos, neg, user_table, item_table):
    tabTu = jnp.swapaxes(user_table, 0, 1)
    tabTi = jnp.swapaxes(item_table, 0, 1)
    outTu, outTp, outTn = _sc_gather3(user, pos, neg, tabTu, tabTi)
    return (
        jnp.swapaxes(outTu, 0, 1),
        jnp.swapaxes(outTp, 0, 1),
        jnp.swapaxes(outTn, 0, 1),
    )


# speedup vs baseline: 1.3008x; 1.3008x over previous
import functools

import jax
import jax.numpy as jnp
from jax import lax
from jax.experimental import pallas as pl
from jax.experimental.pallas import tpu as pltpu
from jax.experimental.pallas import tpu_sc as plsc

DIM = 32
B = 16384
NROWS = 1000000
NCOLS = 7813
NC = 2
NS = 16
NW = NC * NS
BIN_COLS = 245
BIN_ROWS = BIN_COLS * 128
CHUNK_COLS = 12
CHUNK_W = CHUNK_COLS * 128
N_CHUNKS = -(-BIN_COLS // CHUNK_COLS)
MAX_FETCH_BASE = NCOLS * 128 - CHUNK_W
LCAP = 1024
SCAP = 256
SCAN_BLK = 4096
WRING = 16
SCR_LEN = (B + WRING) * DIM
B_PER_W = B // NW

_mesh = plsc.VectorSubcoreMesh(core_axis_name="c", subcore_axis_name="s")


@functools.partial(
    pl.kernel,
    mesh=_mesh,
    out_type=(
        jax.ShapeDtypeStruct((SCR_LEN,), jnp.float32),
        jax.ShapeDtypeStruct((SCR_LEN,), jnp.float32),
        jax.ShapeDtypeStruct((SCR_LEN,), jnp.float32),
    ),
    scratch_types=[
        pltpu.VMEM((DIM, CHUNK_W), jnp.float32),
        pltpu.VMEM((SCAN_BLK,), jnp.int32),
        pltpu.VMEM((LCAP,), jnp.int32),
        pltpu.VMEM((LCAP,), jnp.int32),
        pltpu.VMEM((LCAP,), jnp.int32),
        pltpu.VMEM((LCAP,), jnp.int32),
        pltpu.VMEM((LCAP,), jnp.int32),
        pltpu.VMEM((LCAP,), jnp.int32),
        pltpu.VMEM((SCAP,), jnp.int32),
        pltpu.VMEM((SCAP,), jnp.int32),
        pltpu.VMEM((2 * WRING * DIM,), jnp.float32),
        pltpu.SemaphoreType.DMA((WRING,)),
    ],
    compiler_params=pltpu.CompilerParams(needs_layout_passes=False),
)
def _sc_phase1(user_h, pos_h, neg_h, tabTu_h, tabTi_h,
               scr_u, scr_p, scr_n,
               ring_v, scan_v, lu_ix, lu_pos, lp_ix, lp_pos, ln_ix, ln_pos,
               sub_ix, sub_pos, stage_v, wsems):
    cid = lax.axis_index("c")
    sid = lax.axis_index("s")
    wid = sid * NC + cid

    row_lo = wid * BIN_ROWS
    row_hi = row_lo + BIN_ROWS
    iota16 = lax.iota(jnp.int32, 16)
    d_lo = iota16
    d_hi = iota16 + 16

    def scan_gather(idx_h, list_ix, list_pos):
        cnt = jnp.int32(0)
        for blk in range(B // SCAN_BLK):
            pltpu.sync_copy(idx_h.at[pl.ds(blk * SCAN_BLK, SCAN_BLK)], scan_v)

            def body(i, cnt, blk=blk):
                vec = scan_v[pl.ds(i * 16, 16)]
                mask = (vec >= row_lo) & (vec < row_hi)
                pvec = blk * SCAN_BLK + i * 16 + iota16
                plsc.store_compressed(list_ix.at[pl.ds(cnt, 16)], vec,
                                      mask=mask)
                plsc.store_compressed(list_pos.at[pl.ds(cnt, 16)], pvec,
                                      mask=mask)
                npop = plsc.all_reduce_population_count(mask)
                return cnt + npop[0]

            cnt = lax.fori_loop(0, SCAN_BLK // 16, body, cnt)
        return cnt

    cnt_u = scan_gather(user_h, lu_ix, lu_pos)
    cnt_p = scan_gather(pos_h, lp_ix, lp_pos)
    cnt_n = scan_gather(neg_h, ln_ix, ln_pos)

    col_lo = wid * BIN_COLS

    def extract_list(list_ix, list_pos, cnt, scr, chunk_lo, chunk_hi, base,
                     ggrp):
        def filt(i, scnt):
            vec = list_ix[pl.ds(i * 16, 16)]
            pvec = list_pos[pl.ds(i * 16, 16)]
            valid = (i * 16 + iota16) < cnt
            mask = valid & (vec >= chunk_lo) & (vec < chunk_hi)
            plsc.store_compressed(sub_ix.at[pl.ds(scnt, 16)], vec, mask=mask)
            plsc.store_compressed(sub_pos.at[pl.ds(scnt, 16)], pvec,
                                  mask=mask)
            npop = plsc.all_reduce_population_count(mask)
            return scnt + npop[0]

        scnt = lax.fori_loop(0, LCAP // 16, filt, jnp.int32(0))

        def grp(gi, ggrp):
            vec = sub_ix[pl.ds(gi * 16, 16)]
            pvec = sub_pos[pl.ds(gi * 16, 16)]
            bank = (ggrp % 2) * (WRING * DIM)
            for j in range(16):
                valid = gi * 16 + j < scnt
                ix = vec[j]
                p = pvec[j]
                off16 = jnp.broadcast_to(
                    jnp.where(valid, ix - base, 0), (16,))
                p_eff = jnp.where(valid, p, B + j)
                v_lo = plsc.load_gather(ring_v, [d_lo, off16])
                v_hi = plsc.load_gather(ring_v, [d_hi, off16])

                @pl.when(ggrp >= 2)
                def _(j=j, bank=bank):
                    pltpu.make_async_copy(
                        scr_u.at[pl.ds(0, DIM)],
                        stage_v.at[pl.ds(bank + j * DIM, DIM)],
                        wsems.at[j],
                    ).wait()

                stage_v[pl.ds(bank + j * DIM, 16)] = v_lo
                stage_v[pl.ds(bank + j * DIM + 16, 16)] = v_hi
                pltpu.async_copy(stage_v.at[pl.ds(bank + j * DIM, DIM)],
                                 scr.at[pl.ds(p_eff * DIM, DIM)],
                                 wsems.at[j])
            return ggrp + 1

        n_grps = (scnt + 15) // 16
        return lax.fori_loop(0, n_grps, grp, ggrp)

    ggrp = jnp.int32(0)
    for tab, jobs in (
        (tabTu_h, ((lu_ix, lu_pos, cnt_u, scr_u),)),
        (tabTi_h, ((lp_ix, lp_pos, cnt_p, scr_p),
                   (ln_ix, ln_pos, cnt_n, scr_n))),
    ):
        def chunk_body(c, ggrp, tab=tab, jobs=jobs):
            chunk_lo = (col_lo + c * CHUNK_COLS) * 128
            chunk_hi = chunk_lo + CHUNK_W
            base = pl.multiple_of(jnp.minimum(chunk_lo, MAX_FETCH_BASE), 128)
            pltpu.sync_copy(tab.at[:, pl.ds(base, CHUNK_W)], ring_v)
            for (lix, lpos, lcnt, scr) in jobs:
                ggrp = extract_list(lix, lpos, lcnt, scr, chunk_lo,
                                    chunk_hi, base, ggrp)
            return ggrp

        ggrp = lax.fori_loop(0, N_CHUNKS, chunk_body, ggrp)

    for j in range(16):
        @pl.when(ggrp >= 1)
        def _(j=j):
            pltpu.make_async_copy(
                scr_u.at[pl.ds(0, DIM)],
                stage_v.at[pl.ds(j * DIM, DIM)],
                wsems.at[j],
            ).wait()

        @pl.when(ggrp >= 2)
        def _(j=j):
            pltpu.make_async_copy(
                scr_u.at[pl.ds(0, DIM)],
                stage_v.at[pl.ds(j * DIM, DIM)],
                wsems.at[j],
            ).wait()


@functools.partial(
    pl.kernel,
    mesh=_mesh,
    out_type=(
        jax.ShapeDtypeStruct((DIM, B), jnp.float32),
        jax.ShapeDtypeStruct((DIM, B), jnp.float32),
        jax.ShapeDtypeStruct((DIM, B), jnp.float32),
    ),
    scratch_types=[
        pltpu.VMEM((B_PER_W * DIM,), jnp.float32),
        pltpu.VMEM((DIM, B_PER_W), jnp.float32),
    ],
    compiler_params=pltpu.CompilerParams(needs_layout_passes=False),
)
def _sc_phase2(scr_u, scr_p, scr_n, outTu, outTp, outTn, flat_v, stage_v):
    cid = lax.axis_index("c")
    sid = lax.axis_index("s")
    wid = sid * NC + cid
    base = wid * B_PER_W

    iota16 = lax.iota(jnp.int32, 16)
    d_lo = iota16
    d_hi = iota16 + 16

    for scr, outT in ((scr_u, outTu), (scr_p, outTp), (scr_n, outTn)):
        pltpu.sync_copy(scr.at[pl.ds(base * DIM, B_PER_W * DIM)], flat_v)

        def body(p, _):
            pos16 = jnp.broadcast_to(p, (16,))
            v_lo = plsc.load_gather(flat_v, [p * DIM + d_lo])
            v_hi = plsc.load_gather(flat_v, [p * DIM + d_hi])
            plsc.store_scatter(stage_v, [d_lo, pos16], v_lo)
            plsc.store_scatter(stage_v, [d_hi, pos16], v_hi)
            return 0

        lax.fori_loop(0, B_PER_W, body, 0)
        pltpu.sync_copy(stage_v, outT.at[:, pl.ds(base, B_PER_W)])


def kernel(user, pos, neg, user_table, item_table):
    tabTu = jnp.swapaxes(user_table, 0, 1)
    tabTi = jnp.swapaxes(item_table, 0, 1)
    scr_u, scr_p, scr_n = _sc_phase1(user, pos, neg, tabTu, tabTi)
    outTu, outTp, outTn = _sc_phase2(scr_u, scr_p, scr_n)
    return (
        jnp.swapaxes(outTu, 0, 1),
        jnp.swapaxes(outTp, 0, 1),
        jnp.swapaxes(outTn, 0, 1),
    )


# speedup vs baseline: 1.4550x; 1.1185x over previous
import functools

import jax
import jax.numpy as jnp
from jax import lax
from jax.experimental import pallas as pl
from jax.experimental.pallas import tpu as pltpu
from jax.experimental.pallas import tpu_sc as plsc

DIM = 32
B = 16384
NROWS = 1000000
NCOLS = 7813
NC = 2
NS = 16
NW = NC * NS
BIN_COLS = 245
BIN_ROWS = BIN_COLS * 128
CHUNK_COLS = 12
CHUNK_W = CHUNK_COLS * 128
N_CHUNKS = -(-BIN_COLS // CHUNK_COLS)
MAX_FETCH_BASE = NCOLS * 128 - CHUNK_W
LCAP = 1024
SCAP = 256
SCAN_BLK = 4096
WRING = 16
SCR_LEN = (B + WRING) * DIM
B_PER_W = B // NW

_mesh = plsc.VectorSubcoreMesh(core_axis_name="c", subcore_axis_name="s")


@functools.partial(
    pl.kernel,
    mesh=_mesh,
    out_type=(
        jax.ShapeDtypeStruct((SCR_LEN,), jnp.float32),
        jax.ShapeDtypeStruct((SCR_LEN,), jnp.float32),
        jax.ShapeDtypeStruct((SCR_LEN,), jnp.float32),
    ),
    scratch_types=[
        pltpu.VMEM((DIM, CHUNK_W), jnp.float32),
        pltpu.VMEM((DIM, CHUNK_W), jnp.float32),
        pltpu.SemaphoreType.DMA,
        pltpu.SemaphoreType.DMA,
        pltpu.VMEM((SCAN_BLK,), jnp.int32),
        pltpu.VMEM((LCAP,), jnp.int32),
        pltpu.VMEM((LCAP,), jnp.int32),
        pltpu.VMEM((LCAP,), jnp.int32),
        pltpu.VMEM((LCAP,), jnp.int32),
        pltpu.VMEM((LCAP,), jnp.int32),
        pltpu.VMEM((LCAP,), jnp.int32),
        pltpu.VMEM((SCAP,), jnp.int32),
        pltpu.VMEM((SCAP,), jnp.int32),
        pltpu.VMEM((2 * WRING * DIM,), jnp.float32),
        pltpu.SemaphoreType.DMA((WRING,)),
    ],
    compiler_params=pltpu.CompilerParams(needs_layout_passes=False),
)
def _sc_phase1(user_h, pos_h, neg_h, tabTu_h, tabTi_h,
               scr_u, scr_p, scr_n,
               ring_a, ring_b, fsem_a, fsem_b,
               scan_v, lu_ix, lu_pos, lp_ix, lp_pos, ln_ix, ln_pos,
               sub_ix, sub_pos, stage_v, wsems):
    cid = lax.axis_index("c")
    sid = lax.axis_index("s")
    wid = sid * NC + cid

    row_lo = wid * BIN_ROWS
    row_hi = row_lo + BIN_ROWS
    iota16 = lax.iota(jnp.int32, 16)
    d_lo = iota16
    d_hi = iota16 + 16

    def scan_gather(idx_h, list_ix, list_pos):
        cnt = jnp.int32(0)
        for blk in range(B // SCAN_BLK):
            pltpu.sync_copy(idx_h.at[pl.ds(blk * SCAN_BLK, SCAN_BLK)], scan_v)

            def body(i, cnt, blk=blk):
                vec = scan_v[pl.ds(i * 16, 16)]
                mask = (vec >= row_lo) & (vec < row_hi)
                pvec = blk * SCAN_BLK + i * 16 + iota16
                plsc.store_compressed(list_ix.at[pl.ds(cnt, 16)], vec,
                                      mask=mask)
                plsc.store_compressed(list_pos.at[pl.ds(cnt, 16)], pvec,
                                      mask=mask)
                npop = plsc.all_reduce_population_count(mask)
                return cnt + npop[0]

            cnt = lax.fori_loop(0, SCAN_BLK // 16, body, cnt)
        return cnt

    cnt_u = scan_gather(user_h, lu_ix, lu_pos)
    cnt_p = scan_gather(pos_h, lp_ix, lp_pos)
    cnt_n = scan_gather(neg_h, ln_ix, ln_pos)

    col_lo = wid * BIN_COLS

    def extract_list(ring_v, list_ix, list_pos, cnt, scr, chunk_lo, chunk_hi,
                     base, ggrp):
        def filt(i, scnt):
            vec = list_ix[pl.ds(i * 16, 16)]
            pvec = list_pos[pl.ds(i * 16, 16)]
            valid = (i * 16 + iota16) < cnt
            mask = valid & (vec >= chunk_lo) & (vec < chunk_hi)
            plsc.store_compressed(sub_ix.at[pl.ds(scnt, 16)], vec, mask=mask)
            plsc.store_compressed(sub_pos.at[pl.ds(scnt, 16)], pvec,
                                  mask=mask)
            npop = plsc.all_reduce_population_count(mask)
            return scnt + npop[0]

        scnt = lax.fori_loop(0, LCAP // 16, filt, jnp.int32(0))

        def grp(gi, ggrp):
            vec = sub_ix[pl.ds(gi * 16, 16)]
            pvec = sub_pos[pl.ds(gi * 16, 16)]
            bank = (ggrp % 2) * (WRING * DIM)
            for j in range(16):
                valid = gi * 16 + j < scnt
                ix = vec[j]
                p = pvec[j]
                off16 = jnp.broadcast_to(
                    jnp.where(valid, ix - base, 0), (16,))
                p_eff = jnp.where(valid, p, B + j)
                v_lo = plsc.load_gather(ring_v, [d_lo, off16])
                v_hi = plsc.load_gather(ring_v, [d_hi, off16])

                @pl.when(ggrp >= 2)
                def _(j=j, bank=bank):
                    pltpu.make_async_copy(
                        scr_u.at[pl.ds(0, DIM)],
                        stage_v.at[pl.ds(bank + j * DIM, DIM)],
                        wsems.at[j],
                    ).wait()

                stage_v[pl.ds(bank + j * DIM, 16)] = v_lo
                stage_v[pl.ds(bank + j * DIM + 16, 16)] = v_hi
                pltpu.async_copy(stage_v.at[pl.ds(bank + j * DIM, DIM)],
                                 scr.at[pl.ds(p_eff * DIM, DIM)],
                                 wsems.at[j])
            return ggrp + 1

        n_grps = (scnt + 15) // 16
        return lax.fori_loop(0, n_grps, grp, ggrp)

    N_PAIRS = (N_CHUNKS + 1) // 2

    def chunk_base(c):
        chunk_lo = (col_lo + c * CHUNK_COLS) * 128
        return chunk_lo, pl.multiple_of(
            jnp.minimum(chunk_lo, MAX_FETCH_BASE), 128)

    def fetch(tab, c, ring, fsem):
        _, base = chunk_base(c)
        pltpu.async_copy(tab.at[:, pl.ds(base, CHUNK_W)], ring, fsem)

    def wait_fetch(tab, ring, fsem):
        pltpu.make_async_copy(tab.at[:, pl.ds(0, CHUNK_W)], ring, fsem).wait()

    ggrp = jnp.int32(0)
    for tab, jobs in (
        (tabTu_h, ((lu_ix, lu_pos, cnt_u, scr_u),)),
        (tabTi_h, ((lp_ix, lp_pos, cnt_p, scr_p),
                   (ln_ix, ln_pos, cnt_n, scr_n))),
    ):
        fetch(tab, 0, ring_a, fsem_a)

        def pair_body(ci, ggrp, tab=tab, jobs=jobs):
            c0 = ci * 2
            fetch(tab, c0 + 1, ring_b, fsem_b)
            wait_fetch(tab, ring_a, fsem_a)
            chunk_lo, base = chunk_base(c0)
            for (lix, lpos, lcnt, scr) in jobs:
                ggrp = extract_list(ring_a, lix, lpos, lcnt, scr, chunk_lo,
                                    chunk_lo + CHUNK_W, base, ggrp)

            @pl.when(ci + 1 < N_PAIRS)
            def _():
                fetch(tab, c0 + 2, ring_a, fsem_a)

            wait_fetch(tab, ring_b, fsem_b)
            chunk_lo, base = chunk_base(c0 + 1)
            for (lix, lpos, lcnt, scr) in jobs:
                ggrp = extract_list(ring_b, lix, lpos, lcnt, scr, chunk_lo,
                                    chunk_lo + CHUNK_W, base, ggrp)
            return ggrp

        ggrp = lax.fori_loop(0, N_PAIRS, pair_body, ggrp)

    for j in range(16):
        @pl.when(ggrp >= 1)
        def _(j=j):
            pltpu.make_async_copy(
                scr_u.at[pl.ds(0, DIM)],
                stage_v.at[pl.ds(j * DIM, DIM)],
                wsems.at[j],
            ).wait()

        @pl.when(ggrp >= 2)
        def _(j=j):
            pltpu.make_async_copy(
                scr_u.at[pl.ds(0, DIM)],
                stage_v.at[pl.ds(j * DIM, DIM)],
                wsems.at[j],
            ).wait()


@functools.partial(
    pl.kernel,
    mesh=_mesh,
    out_type=(
        jax.ShapeDtypeStruct((DIM, B), jnp.float32),
        jax.ShapeDtypeStruct((DIM, B), jnp.float32),
        jax.ShapeDtypeStruct((DIM, B), jnp.float32),
    ),
    scratch_types=[
        pltpu.VMEM((B_PER_W * DIM,), jnp.float32),
        pltpu.VMEM((DIM, B_PER_W), jnp.float32),
    ],
    compiler_params=pltpu.CompilerParams(needs_layout_passes=False),
)
def _sc_phase2(scr_u, scr_p, scr_n, outTu, outTp, outTn, flat_v, stage_v):
    cid = lax.axis_index("c")
    sid = lax.axis_index("s")
    wid = sid * NC + cid
    base = wid * B_PER_W

    iota16 = lax.iota(jnp.int32, 16)
    d_lo = iota16
    d_hi = iota16 + 16

    for scr, outT in ((scr_u, outTu), (scr_p, outTp), (scr_n, outTn)):
        pltpu.sync_copy(scr.at[pl.ds(base * DIM, B_PER_W * DIM)], flat_v)

        def body(p, _):
            pos16 = jnp.broadcast_to(p, (16,))
            v_lo = plsc.load_gather(flat_v, [p * DIM + d_lo])
            v_hi = plsc.load_gather(flat_v, [p * DIM + d_hi])
            plsc.store_scatter(stage_v, [d_lo, pos16], v_lo)
            plsc.store_scatter(stage_v, [d_hi, pos16], v_hi)
            return 0

        lax.fori_loop(0, B_PER_W, body, 0)
        pltpu.sync_copy(stage_v, outT.at[:, pl.ds(base, B_PER_W)])


def kernel(user, pos, neg, user_table, item_table):
    tabTu = jnp.swapaxes(user_table, 0, 1)
    tabTi = jnp.swapaxes(item_table, 0, 1)
    scr_u, scr_p, scr_n = _sc_phase1(user, pos, neg, tabTu, tabTi)
    outTu, outTp, outTn = _sc_phase2(scr_u, scr_p, scr_n)
    return (
        jnp.swapaxes(outTu, 0, 1),
        jnp.swapaxes(outTp, 0, 1),
        jnp.swapaxes(outTn, 0, 1),
    )
